# Initial kernel scaffold; baseline (speedup 1.0000x reference)
import jax
import jax.numpy as jnp
from jax.experimental import pallas as pl
from jax.experimental.pallas import tpu as pltpu

_B, _S, _H, _E, _EH, _K = 4, 2048, 2048, 8, 2048, 2
_POOL_TS = 256
_EH_T = 512
_DS_T = 1024


def _pool_kernel(hid_ref, out_ref):
    s = pl.program_id(0)
    part = jnp.sum(hid_ref[...], axis=1)

    @pl.when(s == 0)
    def _():
        out_ref[...] = part

    @pl.when(s > 0)
    def _():
        out_ref[...] += part


def _router_kernel(psum_ref, wr_ref, br_ref, rp_ref, idx_ref, tp_ref):
    logits = jnp.dot(psum_ref[...], wr_ref[...].T,
                     preferred_element_type=jnp.float32) * (1.0 / _S)
    logits = logits + br_ref[...]
    m = jnp.max(logits, axis=1, keepdims=True)
    ex = jnp.exp(logits - m)
    probs = ex / jnp.sum(ex, axis=1, keepdims=True)

    col = jax.lax.broadcasted_iota(jnp.int32, (_B, _E), 1)
    m1 = jnp.max(probs, axis=1, keepdims=True)
    i1 = jnp.min(jnp.where(probs == m1, col, _E), axis=1, keepdims=True)
    masked = jnp.where(col == i1, -jnp.inf, probs)
    m2 = jnp.max(masked, axis=1, keepdims=True)
    i2 = jnp.min(jnp.where(masked == m2, col, _E), axis=1, keepdims=True)

    rp_ref[...] = probs
    idx_ref[...] = jnp.concatenate([i1, i2], axis=1)
    denom = jnp.clip(m1 + m2, 1e-8, None)
    tp_ref[...] = jnp.concatenate([m1, m2], axis=1) / denom


def _gelu(x):
    return 0.5 * x * (1.0 + jax.lax.erf(x * (2.0 ** -0.5)))


def _row_select(mat, e):
    rsel = jax.lax.broadcasted_iota(jnp.int32, (_E, 1), 0) == e
    return jnp.sum(jnp.where(rsel, mat, 0.0), axis=0, keepdims=True)


def _up_kernel(idx_ref, hid_ref, w1_ref, b1_ref, h1_ref):
    b = pl.program_id(0)
    k = pl.program_id(1)
    eh = pl.program_id(2)
    e = idx_ref[b * _K + k]
    h = jnp.dot(hid_ref[0], w1_ref[0], preferred_element_type=jnp.float32)
    b1row = _row_select(b1_ref[:, pl.ds(eh * _EH_T, _EH_T)], e)
    h = _gelu(h + b1row)
    h1_ref[0, 0] = h.astype(jnp.bfloat16)


def _down_kernel(idx_ref, tp_ref, h1_ref, w2_ref, b2_ref, out_ref):
    b = pl.program_id(0)
    k = pl.program_id(2)
    eh = pl.program_id(3)
    e = idx_ref[b * _K + k]
    p = tp_ref[b * _K + k]
    acc = jnp.dot(h1_ref[0, 0], w2_ref[0],
                  preferred_element_type=jnp.float32) * p
    alpha = jnp.where(eh == 0, p, 0.0)
    acc = acc + alpha * _row_select(b2_ref, e)

    first = jnp.logical_and(k == 0, eh == 0)

    @pl.when(first)
    def _():
        out_ref[0] = acc

    @pl.when(jnp.logical_not(first))
    def _():
        out_ref[0] += acc


def kernel(hidden, Wr, br, W1, b1, W2, b2):
    f32 = jnp.float32

    pooled_sum = pl.pallas_call(
        _pool_kernel,
        grid=(_S // _POOL_TS,),
        in_specs=[pl.BlockSpec((_B, _POOL_TS, _H), lambda s: (0, s, 0))],
        out_specs=pl.BlockSpec((_B, _H), lambda s: (0, 0)),
        out_shape=jax.ShapeDtypeStruct((_B, _H), f32),
    )(hidden)

    router_probs, topk_idx, topk_probs = pl.pallas_call(
        _router_kernel,
        out_shape=(
            jax.ShapeDtypeStruct((_B, _E), f32),
            jax.ShapeDtypeStruct((_B, _K), jnp.int32),
            jax.ShapeDtypeStruct((_B, _K), f32),
        ),
    )(pooled_sum, Wr, br.reshape(1, _E))

    idx_flat = topk_idx.reshape(-1)
    tp_flat = topk_probs.reshape(-1)

    up_grid = pltpu.PrefetchScalarGridSpec(
        num_scalar_prefetch=1,
        grid=(_B, _K, _EH // _EH_T),
        in_specs=[
            pl.BlockSpec((1, _S, _H), lambda b, k, eh, idx: (b, 0, 0)),
            pl.BlockSpec((1, _H, _EH_T),
                         lambda b, k, eh, idx: (idx[b * _K + k], 0, eh)),
            pl.BlockSpec((_E, _EH), lambda b, k, eh, idx: (0, 0)),
        ],
        out_specs=pl.BlockSpec((1, 1, _S, _EH_T),
                               lambda b, k, eh, idx: (b, k, 0, eh)),
    )
    h1 = pl.pallas_call(
        _up_kernel,
        grid_spec=up_grid,
        out_shape=jax.ShapeDtypeStruct((_B, _K, _S, _EH), jnp.bfloat16),
        compiler_params=pltpu.CompilerParams(
            dimension_semantics=("arbitrary", "arbitrary", "arbitrary")),
    )(idx_flat, hidden, W1, b1)

    down_grid = pltpu.PrefetchScalarGridSpec(
        num_scalar_prefetch=2,
        grid=(_B, _S // _DS_T, _K, _EH // _EH_T),
        in_specs=[
            pl.BlockSpec((1, 1, _DS_T, _EH_T),
                         lambda b, s, k, eh, idx, tp: (b, k, s, eh)),
            pl.BlockSpec((1, _EH_T, _H),
                         lambda b, s, k, eh, idx, tp: (idx[b * _K + k], eh, 0)),
            pl.BlockSpec((_E, _H), lambda b, s, k, eh, idx, tp: (0, 0)),
        ],
        out_specs=pl.BlockSpec((1, _DS_T, _H),
                               lambda b, s, k, eh, idx, tp: (b, s, 0)),
    )
    delta = pl.pallas_call(
        _down_kernel,
        grid_spec=down_grid,
        out_shape=jax.ShapeDtypeStruct((_B, _S, _H), f32),
        compiler_params=pltpu.CompilerParams(
            dimension_semantics=("arbitrary", "arbitrary", "arbitrary",
                                 "arbitrary")),
    )(idx_flat, tp_flat, h1, W2, b2)

    return delta, topk_idx, topk_probs, router_probs


# speedup vs baseline: 3.4230x; 3.4230x over previous
import jax
import jax.numpy as jnp
from jax.experimental import pallas as pl
from jax.experimental.pallas import tpu as pltpu

_B, _S, _H, _E, _EH, _K = 4, 2048, 2048, 8, 2048, 2
_POOL_TS = 256
_EH_T = 512
_DS_T = 1024


def _pool_kernel(hid_ref, out_ref):
    s = pl.program_id(0)
    part = jnp.sum(hid_ref[...], axis=1)

    @pl.when(s == 0)
    def _():
        out_ref[...] = part

    @pl.when(s > 0)
    def _():
        out_ref[...] += part


def _router_kernel(psum_ref, wr_ref, br_ref, rp_ref, idx_ref, tp_ref):
    logits = jnp.dot(psum_ref[...], wr_ref[...].T,
                     preferred_element_type=jnp.float32) * (1.0 / _S)
    logits = logits + br_ref[...]
    m = jnp.max(logits, axis=1, keepdims=True)
    ex = jnp.exp(logits - m)
    probs = ex / jnp.sum(ex, axis=1, keepdims=True)

    col = jax.lax.broadcasted_iota(jnp.int32, (_B, _E), 1)
    m1 = jnp.max(probs, axis=1, keepdims=True)
    i1 = jnp.min(jnp.where(probs == m1, col, _E), axis=1, keepdims=True)
    masked = jnp.where(col == i1, -jnp.inf, probs)
    m2 = jnp.max(masked, axis=1, keepdims=True)
    i2 = jnp.min(jnp.where(masked == m2, col, _E), axis=1, keepdims=True)

    rp_ref[...] = probs
    idx_ref[...] = jnp.concatenate([i1, i2], axis=1)
    denom = jnp.clip(m1 + m2, 1e-8, None)
    tp_ref[...] = jnp.concatenate([m1, m2], axis=1) / denom


def _gelu(x):
    return 0.5 * x * (1.0 + jax.lax.erf(x * (2.0 ** -0.5)))


def _row_select(mat, e):
    rsel = jax.lax.broadcasted_iota(jnp.int32, (_E, 1), 0) == e
    return jnp.sum(jnp.where(rsel, mat, 0.0), axis=0, keepdims=True)


def _up_kernel(idx_ref, hid_ref, w1_ref, b1_ref, h1_ref):
    b = pl.program_id(0)
    k = pl.program_id(1)
    eh = pl.program_id(2)
    e = idx_ref[b * _K + k]
    h = jnp.dot(hid_ref[0], w1_ref[0], preferred_element_type=jnp.float32)
    b1row = _row_select(b1_ref[:, pl.ds(eh * _EH_T, _EH_T)], e)
    h = _gelu(h + b1row)
    h1_ref[0, 0] = h.astype(jnp.bfloat16)


def _down_kernel(idx_ref, tp_ref, h1_ref, w2_ref, b2_ref, out_ref):
    b = pl.program_id(0)
    k = pl.program_id(2)
    eh = pl.program_id(3)
    e = idx_ref[b * _K + k]
    p = tp_ref[b * _K + k]
    acc = jnp.dot(h1_ref[0, 0], w2_ref[0],
                  preferred_element_type=jnp.float32) * p
    alpha = jnp.where(eh == 0, p, 0.0)
    acc = acc + alpha * _row_select(b2_ref[...], e)

    first = jnp.logical_and(k == 0, eh == 0)

    @pl.when(first)
    def _():
        out_ref[0] = acc

    @pl.when(jnp.logical_not(first))
    def _():
        out_ref[0] += acc


def kernel(hidden, Wr, br, W1, b1, W2, b2):
    f32 = jnp.float32

    pooled_sum = pl.pallas_call(
        _pool_kernel,
        grid=(_S // _POOL_TS,),
        in_specs=[pl.BlockSpec((_B, _POOL_TS, _H), lambda s: (0, s, 0))],
        out_specs=pl.BlockSpec((_B, _H), lambda s: (0, 0)),
        out_shape=jax.ShapeDtypeStruct((_B, _H), f32),
    )(hidden)

    router_probs, topk_idx, topk_probs = pl.pallas_call(
        _router_kernel,
        out_shape=(
            jax.ShapeDtypeStruct((_B, _E), f32),
            jax.ShapeDtypeStruct((_B, _K), jnp.int32),
            jax.ShapeDtypeStruct((_B, _K), f32),
        ),
    )(pooled_sum, Wr, br.reshape(1, _E))

    idx_flat = topk_idx.reshape(-1)
    tp_flat = topk_probs.reshape(-1)

    up_grid = pltpu.PrefetchScalarGridSpec(
        num_scalar_prefetch=1,
        grid=(_B, _K, _EH // _EH_T),
        in_specs=[
            pl.BlockSpec((1, _S, _H), lambda b, k, eh, idx: (b, 0, 0)),
            pl.BlockSpec((1, _H, _EH_T),
                         lambda b, k, eh, idx: (idx[b * _K + k], 0, eh)),
            pl.BlockSpec((_E, _EH), lambda b, k, eh, idx: (0, 0)),
        ],
        out_specs=pl.BlockSpec((1, 1, _S, _EH_T),
                               lambda b, k, eh, idx: (b, k, 0, eh)),
    )
    h1 = pl.pallas_call(
        _up_kernel,
        grid_spec=up_grid,
        out_shape=jax.ShapeDtypeStruct((_B, _K, _S, _EH), jnp.bfloat16),
        compiler_params=pltpu.CompilerParams(
            dimension_semantics=("arbitrary", "arbitrary", "arbitrary")),
    )(idx_flat, hidden, W1, b1)

    down_grid = pltpu.PrefetchScalarGridSpec(
        num_scalar_prefetch=2,
        grid=(_B, _S // _DS_T, _K, _EH // _EH_T),
        in_specs=[
            pl.BlockSpec((1, 1, _DS_T, _EH_T),
                         lambda b, s, k, eh, idx, tp: (b, k, s, eh)),
            pl.BlockSpec((1, _EH_T, _H),
                         lambda b, s, k, eh, idx, tp: (idx[b * _K + k], eh, 0)),
            pl.BlockSpec((_E, _H), lambda b, s, k, eh, idx, tp: (0, 0)),
        ],
        out_specs=pl.BlockSpec((1, _DS_T, _H),
                               lambda b, s, k, eh, idx, tp: (b, s, 0)),
    )
    delta = pl.pallas_call(
        _down_kernel,
        grid_spec=down_grid,
        out_shape=jax.ShapeDtypeStruct((_B, _S, _H), f32),
        compiler_params=pltpu.CompilerParams(
            dimension_semantics=("arbitrary", "arbitrary", "arbitrary",
                                 "arbitrary")),
    )(idx_flat, tp_flat, h1, W2, b2)

    return delta, topk_idx, topk_probs, router_probs
